# baseline (speedup 1.0000x reference)
import jax
import jax.numpy as jnp
from jax import lax
from jax.experimental import pallas as pl
from jax.experimental.pallas import tpu as pltpu
from jax.experimental.pallas import tpu_sc as plsc

NC = 2
NS = 16
NW = NC * NS
L = 16

ROWS = 100000
DIM = 128
BATCH = 16384

GRANULE = 16
NGRAN = ROWS // GRANULE
GPW = NGRAN // NW
EXTRA = NGRAN - GPW * NW
CHUNK_G = 8
CHUNK_ROWS = CHUNK_G * GRANULE
VPC = CHUNK_ROWS // L
N_FULL_CHUNKS = GPW // CHUNK_G
N_PAIRS = N_FULL_CHUNKS // 2
BPW = BATCH // NW
PREV_CHUNKS = BPW // CHUNK_ROWS
MIN_ROWS = GPW * GRANULE
MAX_CHUNKS = BATCH // CHUNK_ROWS + 1


def _body(mem_hbm, lu_hbm, val_hbm, ts_hbm, nodes_hbm,
          newmem_hbm, newlu_hbm, prev_hbm,
          nodes_v, ts_v, mn_v, mi_v, mnc_v, win_v, lu_v,
          rows_a, rows_b, sem_a, sem_b, sem_oa, sem_ob):
    i32 = jnp.int32
    w = lax.axis_index("s") * NC + lax.axis_index("c")
    base_g = GPW * w + jnp.minimum(w, EXTRA)
    n_g = GPW + jnp.where(w < EXTRA, 1, 0)
    base_row = base_g * GRANULE
    n_rows = n_g * GRANULE
    iota16 = lax.iota(i32, L)

    c_nodes = pltpu.async_copy(nodes_hbm, nodes_v, sem_a)
    c_ts = pltpu.async_copy(ts_hbm, ts_v, sem_b)
    c_lu = pltpu.async_copy(lu_hbm.at[pl.ds(base_row, MIN_ROWS)],
                            lu_v.at[pl.ds(0, MIN_ROWS)], sem_oa)
    c_nodes.wait()
    c_ts.wait()
    c_lu.wait()

    @pl.when(n_g > GPW)
    def _():
        pltpu.async_copy(lu_hbm.at[pl.ds(base_row + MIN_ROWS, GRANULE)],
                         lu_v.at[pl.ds(MIN_ROWS, GRANULE)], sem_oa).wait()

    bbase = BPW * w

    def prev_in(k, buf, sem):
        idx = nodes_v.at[pl.ds(bbase + CHUNK_ROWS * k, CHUNK_ROWS)]
        return pltpu.async_copy(mem_hbm.at[idx], buf, sem)

    def prev_out(k, buf, sem):
        return pltpu.async_copy(
            buf, prev_hbm.at[pl.ds(bbase + CHUNK_ROWS * k, CHUNK_ROWS)], sem)

    g0 = prev_in(0, rows_a, sem_a)
    g0.wait()
    g1 = prev_in(1, rows_b, sem_b)
    o0 = prev_out(0, rows_a, sem_oa)
    g1.wait()
    o1 = prev_out(1, rows_b, sem_ob)
    o0.wait()
    g2 = prev_in(2, rows_a, sem_a)
    g2.wait()
    o1.wait()
    g3 = prev_in(3, rows_b, sem_b)
    o2 = prev_out(2, rows_a, sem_oa)
    g3.wait()
    o3 = prev_out(3, rows_b, sem_ob)
    o2.wait()
    o3.wait()

    c_main = pltpu.async_copy(mem_hbm.at[pl.ds(base_row, MIN_ROWS)],
                              newmem_hbm.at[pl.ds(base_row, MIN_ROWS)], sem_a)

    @pl.when(n_g > GPW)
    def _():
        pltpu.async_copy(
            mem_hbm.at[pl.ds(base_row + MIN_ROWS, GRANULE)],
            newmem_hbm.at[pl.ds(base_row + MIN_ROWS, GRANULE)], sem_b).wait()

    c_main.wait()

    def scan_step(k, off):
        n16 = nodes_v[pl.ds(L * k, L)]
        m = (n16 >= base_row) & (n16 < base_row + n_rows)
        mi = m.astype(i32)
        tgt = jnp.maximum(off + plsc.cumsum(mi) - 1, 0)
        plsc.store_scatter(mn_v, [tgt], n16, mask=m)
        plsc.store_scatter(mi_v, [tgt], iota16 + L * k, mask=m)
        return off + jnp.sum(mi)

    nmatch = lax.fori_loop(0, BATCH // L, scan_step, jnp.int32(0))

    @pl.when(nmatch > 0)
    def _():
        lastn = plsc.load_gather(mn_v, [jnp.full((L,), nmatch - 1, i32)])
        lasti = plsc.load_gather(mi_v, [jnp.full((L,), nmatch - 1, i32)])
        for t in range(VPC):
            plsc.store_scatter(mn_v, [nmatch + L * t + iota16], lastn)
            plsc.store_scatter(mi_v, [nmatch + L * t + iota16], lasti)

    n_vregs = (nmatch + L - 1) // L
    n_chunks = (nmatch + CHUNK_ROWS - 1) // CHUNK_ROWS

    def pass_a(k, _):
        n16 = mn_v[pl.ds(L * k, L)]
        i16 = mi_v[pl.ds(L * k, L)]
        _, last = plsc.scan_count(n16)
        plsc.store_scatter(win_v, [n16 - base_row], i16, mask=last)
        return 0

    lax.fori_loop(0, n_vregs, pass_a, 0)

    def prep(k, _):
        n16 = mn_v[pl.ds(L * k, L)]
        loc = n16 - base_row
        wv = plsc.load_gather(win_v, [loc])
        mi_v[pl.ds(L * k, L)] = wv
        plsc.store_scatter(
            mnc_v, [jnp.full((L,), k // VPC, i32), (k % VPC) * L + iota16],
            n16)
        tsv = plsc.load_gather(ts_v, [wv])
        plsc.store_scatter(lu_v, [loc], tsv)
        return 0

    lax.fori_loop(0, n_chunks * VPC, prep, 0)

    def pass_b(j, _):
        pltpu.async_copy(
            val_hbm.at[mi_v.at[pl.ds(CHUNK_ROWS * j, CHUNK_ROWS)]],
            rows_a, sem_a).wait()
        pltpu.async_copy(rows_a, newmem_hbm.at[mnc_v.at[j]], sem_oa).wait()
        return 0

    lax.fori_loop(0, n_chunks, pass_b, 0)

    pltpu.async_copy(lu_v.at[pl.ds(0, MIN_ROWS)],
                     newlu_hbm.at[pl.ds(base_row, MIN_ROWS)], sem_oa).wait()

    @pl.when(n_g > GPW)
    def _():
        pltpu.async_copy(lu_v.at[pl.ds(MIN_ROWS, GRANULE)],
                         newlu_hbm.at[pl.ds(base_row + MIN_ROWS, GRANULE)],
                         sem_oa).wait()


@jax.jit
def kernel(memory, last_update, values, ts, nodes):
    mesh = plsc.VectorSubcoreMesh(core_axis_name="c", subcore_axis_name="s",
                                  num_cores=NC, num_subcores=NS)
    out_type = (
        jax.ShapeDtypeStruct((ROWS, DIM), jnp.float32),
        jax.ShapeDtypeStruct((ROWS,), jnp.float32),
        jax.ShapeDtypeStruct((BATCH, DIM), jnp.float32),
    )
    scratch = [
        pltpu.VMEM((BATCH,), jnp.int32),
        pltpu.VMEM((BATCH,), jnp.float32),
        pltpu.VMEM((BATCH + CHUNK_ROWS,), jnp.int32),
        pltpu.VMEM((BATCH + CHUNK_ROWS,), jnp.int32),
        pltpu.VMEM((MAX_CHUNKS, CHUNK_ROWS), jnp.int32),
        pltpu.VMEM(((GPW + 1) * GRANULE,), jnp.int32),
        pltpu.VMEM(((GPW + 1) * GRANULE,), jnp.float32),
        pltpu.VMEM((CHUNK_ROWS, DIM), jnp.float32),
        pltpu.VMEM((CHUNK_ROWS, DIM), jnp.float32),
        pltpu.SemaphoreType.DMA,
        pltpu.SemaphoreType.DMA,
        pltpu.SemaphoreType.DMA,
        pltpu.SemaphoreType.DMA,
    ]
    cp = pltpu.CompilerParams(needs_layout_passes=False)
    f = pl.kernel(_body, out_type=out_type, mesh=mesh, scratch_types=scratch,
                  compiler_params=cp)
    return f(memory, last_update, values.astype(jnp.float32),
             ts.astype(jnp.float32), nodes.astype(jnp.int32))


# speedup vs baseline: 12.6131x; 12.6131x over previous
import jax
import jax.numpy as jnp
from jax import lax
from jax.experimental import pallas as pl
from jax.experimental.pallas import tpu as pltpu
from jax.experimental.pallas import tpu_sc as plsc

NC = 2
NS = 16
NW = NC * NS
L = 16

ROWS = 100000
DIM = 128
BATCH = 16384

GRANULE = 16
NGRAN = ROWS // GRANULE
GPW = NGRAN // NW
EXTRA = NGRAN - GPW * NW
CHUNK_G = 8
CHUNK_ROWS = CHUNK_G * GRANULE
VPC = CHUNK_ROWS // L
N_FULL_CHUNKS = GPW // CHUNK_G
N_PAIRS = N_FULL_CHUNKS // 2
BPW = BATCH // NW
PREV_CHUNKS = BPW // CHUNK_ROWS
MIN_ROWS = GPW * GRANULE
MAX_OWN = (GPW + 1) * GRANULE
MAX_CHUNKS = (MAX_OWN + CHUNK_ROWS - 1) // CHUNK_ROWS + 1
UNROLL = 4


def _body(mem_hbm, lu_hbm, val_hbm, ts_hbm, nodes_hbm,
          newmem_hbm, newlu_hbm, prev_hbm,
          nodes_v, ts_v, mn_v, mi_v, mnc_v, win_v, lu_v,
          rows_a, rows_b, sem_a, sem_b, sem_oa, sem_ob):
    i32 = jnp.int32
    w = lax.axis_index("s") * NC + lax.axis_index("c")
    base_g = GPW * w + jnp.minimum(w, EXTRA)
    n_g = GPW + jnp.where(w < EXTRA, 1, 0)
    base_row = base_g * GRANULE
    n_rows = n_g * GRANULE
    iota16 = lax.iota(i32, L)

    c_nodes = pltpu.async_copy(nodes_hbm, nodes_v, sem_a)
    c_ts = pltpu.async_copy(ts_hbm, ts_v, sem_b)
    c_lu = pltpu.async_copy(lu_hbm.at[pl.ds(base_row, MIN_ROWS)],
                            lu_v.at[pl.ds(0, MIN_ROWS)], sem_oa)
    c_nodes.wait()
    c_ts.wait()
    c_lu.wait()

    @pl.when(n_g > GPW)
    def _():
        pltpu.async_copy(lu_hbm.at[pl.ds(base_row + MIN_ROWS, GRANULE)],
                         lu_v.at[pl.ds(MIN_ROWS, GRANULE)], sem_oa).wait()

    bbase = BPW * w

    def prev_in(k, buf, sem):
        idx = nodes_v.at[pl.ds(bbase + CHUNK_ROWS * k, CHUNK_ROWS)]
        return pltpu.async_copy(mem_hbm.at[idx], buf, sem)

    def prev_out(k, buf, sem):
        return pltpu.async_copy(
            buf, prev_hbm.at[pl.ds(bbase + CHUNK_ROWS * k, CHUNK_ROWS)], sem)

    g0 = prev_in(0, rows_a, sem_a)
    g0.wait()
    g1 = prev_in(1, rows_b, sem_b)
    o0 = prev_out(0, rows_a, sem_oa)
    g1.wait()
    o1 = prev_out(1, rows_b, sem_ob)
    o0.wait()
    g2 = prev_in(2, rows_a, sem_a)
    g2.wait()
    o1.wait()
    g3 = prev_in(3, rows_b, sem_b)
    o2 = prev_out(2, rows_a, sem_oa)
    g3.wait()
    o3 = prev_out(3, rows_b, sem_ob)
    o2.wait()
    o3.wait()

    def cp_in(r, buf, sem):
        return pltpu.async_copy(mem_hbm.at[pl.ds(r, CHUNK_ROWS)], buf, sem)

    def cp_out(r, buf, sem):
        return pltpu.async_copy(buf, newmem_hbm.at[pl.ds(r, CHUNK_ROWS)], sem)

    cp_in(base_row, rows_a, sem_a)

    def copy_pair(p, _):
        r0 = base_row + CHUNK_ROWS * 2 * p
        r1 = r0 + CHUNK_ROWS
        pltpu.make_async_copy(mem_hbm.at[pl.ds(r0, CHUNK_ROWS)], rows_a,
                              sem_a).wait()

        @pl.when(p > 0)
        def _():
            pltpu.make_async_copy(
                rows_b, newmem_hbm.at[pl.ds(r0 - CHUNK_ROWS, CHUNK_ROWS)],
                sem_ob).wait()

        cp_in(r1, rows_b, sem_b)
        cp_out(r0, rows_a, sem_oa)
        pltpu.make_async_copy(mem_hbm.at[pl.ds(r1, CHUNK_ROWS)], rows_b,
                              sem_b).wait()
        pltpu.make_async_copy(rows_a, newmem_hbm.at[pl.ds(r0, CHUNK_ROWS)],
                              sem_oa).wait()

        @pl.when(p < N_PAIRS - 1)
        def _():
            cp_in(r1 + CHUNK_ROWS, rows_a, sem_a)

        cp_out(r1, rows_b, sem_ob)
        return 0

    lax.fori_loop(0, N_PAIRS, copy_pair, 0)
    pltpu.make_async_copy(
        rows_b,
        newmem_hbm.at[pl.ds(base_row + CHUNK_ROWS * (N_FULL_CHUNKS - 1),
                            CHUNK_ROWS)],
        sem_ob).wait()

    def copy_tail(t, _):
        r = base_row + N_FULL_CHUNKS * CHUNK_ROWS + GRANULE * t
        pltpu.async_copy(mem_hbm.at[pl.ds(r, GRANULE)],
                         rows_a.at[pl.ds(0, GRANULE)], sem_a).wait()
        pltpu.async_copy(rows_a.at[pl.ds(0, GRANULE)],
                         newmem_hbm.at[pl.ds(r, GRANULE)], sem_a).wait()
        return 0

    lax.fori_loop(0, n_g - N_FULL_CHUNKS * CHUNK_G, copy_tail, 0)

    def win_init(t, _):
        win_v[pl.ds(L * t, L)] = jnp.full((L,), -1, i32)
        return 0

    lax.fori_loop(0, n_g, win_init, 0)

    def win_scan(q, _):
        for u in range(UNROLL):
            k = UNROLL * q + u
            n16 = nodes_v[pl.ds(L * k, L)]
            loc = n16 - base_row
            m = (loc >= 0) & (loc < n_rows)
            n_sel = jnp.where(m, n16, -1)
            _, last = plsc.scan_count(n_sel)
            plsc.store_scatter(win_v, [jnp.clip(loc, 0, n_rows - 1)],
                               iota16 + L * k, mask=m & last)
        return 0

    lax.fori_loop(0, BATCH // L // UNROLL, win_scan, 0)

    def comp(t, off):
        wv = win_v[pl.ds(L * t, L)]
        m = wv >= 0
        mi = m.astype(i32)
        tgt = jnp.maximum(off + plsc.cumsum(mi) - 1, 0)
        plsc.store_scatter(mi_v, [tgt], wv, mask=m)
        plsc.store_scatter(mn_v, [tgt], base_row + L * t + iota16, mask=m)
        return off + jnp.sum(mi)

    nuniq = lax.fori_loop(0, n_g, comp, jnp.int32(0))
    n_chunks = (nuniq + CHUNK_ROWS - 1) // CHUNK_ROWS

    @pl.when(nuniq > 0)
    def _():
        lastn = plsc.load_gather(mn_v, [jnp.full((L,), nuniq - 1, i32)])
        lasti = plsc.load_gather(mi_v, [jnp.full((L,), nuniq - 1, i32)])
        for t in range(VPC):
            plsc.store_scatter(mn_v, [nuniq + L * t + iota16], lastn)
            plsc.store_scatter(mi_v, [nuniq + L * t + iota16], lasti)

    def prep(k, _):
        n16 = mn_v[pl.ds(L * k, L)]
        wv = mi_v[pl.ds(L * k, L)]
        plsc.store_scatter(
            mnc_v, [jnp.full((L,), k // VPC, i32), (k % VPC) * L + iota16],
            n16)
        tsv = plsc.load_gather(ts_v, [wv])
        plsc.store_scatter(lu_v, [n16 - base_row], tsv)
        return 0

    lax.fori_loop(0, n_chunks * VPC, prep, 0)

    def pass_b(j, _):
        pltpu.async_copy(
            val_hbm.at[mi_v.at[pl.ds(CHUNK_ROWS * j, CHUNK_ROWS)]],
            rows_a, sem_a).wait()
        pltpu.async_copy(rows_a, newmem_hbm.at[mnc_v.at[j]], sem_oa).wait()
        return 0

    lax.fori_loop(0, n_chunks, pass_b, 0)

    pltpu.async_copy(lu_v.at[pl.ds(0, MIN_ROWS)],
                     newlu_hbm.at[pl.ds(base_row, MIN_ROWS)], sem_oa).wait()

    @pl.when(n_g > GPW)
    def _():
        pltpu.async_copy(lu_v.at[pl.ds(MIN_ROWS, GRANULE)],
                         newlu_hbm.at[pl.ds(base_row + MIN_ROWS, GRANULE)],
                         sem_ob).wait()


@jax.jit
def kernel(memory, last_update, values, ts, nodes):
    mesh = plsc.VectorSubcoreMesh(core_axis_name="c", subcore_axis_name="s",
                                  num_cores=NC, num_subcores=NS)
    out_type = (
        jax.ShapeDtypeStruct((ROWS, DIM), jnp.float32),
        jax.ShapeDtypeStruct((ROWS,), jnp.float32),
        jax.ShapeDtypeStruct((BATCH, DIM), jnp.float32),
    )
    scratch = [
        pltpu.VMEM((BATCH,), jnp.int32),
        pltpu.VMEM((BATCH,), jnp.float32),
        pltpu.VMEM((MAX_OWN + CHUNK_ROWS,), jnp.int32),
        pltpu.VMEM((MAX_OWN + CHUNK_ROWS,), jnp.int32),
        pltpu.VMEM((MAX_CHUNKS, CHUNK_ROWS), jnp.int32),
        pltpu.VMEM((MAX_OWN,), jnp.int32),
        pltpu.VMEM((MAX_OWN,), jnp.float32),
        pltpu.VMEM((CHUNK_ROWS, DIM), jnp.float32),
        pltpu.VMEM((CHUNK_ROWS, DIM), jnp.float32),
        pltpu.SemaphoreType.DMA,
        pltpu.SemaphoreType.DMA,
        pltpu.SemaphoreType.DMA,
        pltpu.SemaphoreType.DMA,
    ]
    cp = pltpu.CompilerParams(needs_layout_passes=False)
    f = pl.kernel(_body, out_type=out_type, mesh=mesh, scratch_types=scratch,
                  compiler_params=cp)
    return f(memory, last_update, values.astype(jnp.float32),
             ts.astype(jnp.float32), nodes.astype(jnp.int32))


# speedup vs baseline: 13.8290x; 1.0964x over previous
import jax
import jax.numpy as jnp
from jax import lax
from jax.experimental import pallas as pl
from jax.experimental.pallas import tpu as pltpu
from jax.experimental.pallas import tpu_sc as plsc

NC = 2
NS = 16
NW = NC * NS
L = 16

ROWS = 100000
DIM = 128
BATCH = 16384

GRANULE = 16
NGRAN = ROWS // GRANULE
GPW = NGRAN // NW
EXTRA = NGRAN - GPW * NW
CHUNK_G = 8
CHUNK_ROWS = CHUNK_G * GRANULE
VPC = CHUNK_ROWS // L
CP_G = 16
CP_ROWS = CP_G * GRANULE
N_CP = GPW // CP_G
N_CP_PAIRS = N_CP // 2
BPW = BATCH // NW
PREV_CHUNKS = BPW // CHUNK_ROWS
MIN_ROWS = GPW * GRANULE
MAX_OWN = (GPW + 1) * GRANULE
MAX_CHUNKS = (MAX_OWN + CHUNK_ROWS - 1) // CHUNK_ROWS + 1
UNROLL = 4


def _body(mem_hbm, lu_hbm, val_hbm, ts_hbm, nodes_hbm,
          newmem_hbm, newlu_hbm, prev_hbm,
          nodes_v, ts_v, mn_v, mi_v, mnc_v, win_v, lu_v,
          rows_a, rows_b, sem_a, sem_b, sem_oa, sem_ob, sem_s):
    i32 = jnp.int32
    w = lax.axis_index("s") * NC + lax.axis_index("c")
    base_g = GPW * w + jnp.minimum(w, EXTRA)
    n_g = GPW + jnp.where(w < EXTRA, 1, 0)
    base_row = base_g * GRANULE
    n_rows = n_g * GRANULE
    iota16 = lax.iota(i32, L)

    c_nodes = pltpu.async_copy(nodes_hbm, nodes_v, sem_a)
    c_ts = pltpu.async_copy(ts_hbm, ts_v, sem_s)
    c_lu = pltpu.async_copy(lu_hbm.at[pl.ds(base_row, MIN_ROWS)],
                            lu_v.at[pl.ds(0, MIN_ROWS)], sem_s)
    lu_tail = jnp.minimum(base_row + MIN_ROWS, ROWS - GRANULE)
    c_lut = pltpu.async_copy(lu_hbm.at[pl.ds(lu_tail, GRANULE)],
                             lu_v.at[pl.ds(MIN_ROWS, GRANULE)], sem_s)
    c_nodes.wait()

    bbase = BPW * w

    def prev_in(k, buf, sem):
        idx = nodes_v.at[pl.ds(bbase + CHUNK_ROWS * k, CHUNK_ROWS)]
        return pltpu.async_copy(mem_hbm.at[idx], buf.at[pl.ds(0, CHUNK_ROWS)], sem)

    def prev_out(k, buf, sem):
        return pltpu.async_copy(
            buf.at[pl.ds(0, CHUNK_ROWS)],
            prev_hbm.at[pl.ds(bbase + CHUNK_ROWS * k, CHUNK_ROWS)], sem)

    g0 = prev_in(0, rows_a, sem_a)
    g0.wait()
    g1 = prev_in(1, rows_b, sem_b)
    o0 = prev_out(0, rows_a, sem_oa)
    g1.wait()
    o1 = prev_out(1, rows_b, sem_ob)
    o0.wait()
    g2 = prev_in(2, rows_a, sem_a)
    g2.wait()
    o1.wait()
    g3 = prev_in(3, rows_b, sem_b)
    o2 = prev_out(2, rows_a, sem_oa)
    g3.wait()
    o3 = prev_out(3, rows_b, sem_ob)
    o2.wait()
    o3.wait()

    def cp_in(r, buf, sem):
        return pltpu.async_copy(mem_hbm.at[pl.ds(r, CP_ROWS)], buf, sem)

    def cp_out(r, buf, sem):
        return pltpu.async_copy(buf, newmem_hbm.at[pl.ds(r, CP_ROWS)], sem)

    cp_in(base_row, rows_a, sem_a)

    def copy_pair(p, _):
        r0 = base_row + CP_ROWS * 2 * p
        r1 = r0 + CP_ROWS
        pltpu.make_async_copy(mem_hbm.at[pl.ds(r0, CP_ROWS)], rows_a,
                              sem_a).wait()

        @pl.when(p > 0)
        def _():
            pltpu.make_async_copy(
                rows_b, newmem_hbm.at[pl.ds(r0 - CP_ROWS, CP_ROWS)],
                sem_ob).wait()

        cp_in(r1, rows_b, sem_b)
        cp_out(r0, rows_a, sem_oa)
        pltpu.make_async_copy(mem_hbm.at[pl.ds(r1, CP_ROWS)], rows_b,
                              sem_b).wait()
        pltpu.make_async_copy(rows_a, newmem_hbm.at[pl.ds(r0, CP_ROWS)],
                              sem_oa).wait()

        @pl.when(p < N_CP_PAIRS - 1)
        def _():
            cp_in(r1 + CP_ROWS, rows_a, sem_a)

        cp_out(r1, rows_b, sem_ob)
        return 0

    lax.fori_loop(0, N_CP_PAIRS, copy_pair, 0)
    pltpu.make_async_copy(
        rows_b,
        newmem_hbm.at[pl.ds(base_row + CP_ROWS * (N_CP - 1), CP_ROWS)],
        sem_ob).wait()

    def copy_tail(t, _):
        r = base_row + N_CP * CP_ROWS + GRANULE * t
        pltpu.async_copy(mem_hbm.at[pl.ds(r, GRANULE)],
                         rows_a.at[pl.ds(0, GRANULE)], sem_a).wait()
        pltpu.async_copy(rows_a.at[pl.ds(0, GRANULE)],
                         newmem_hbm.at[pl.ds(r, GRANULE)], sem_a).wait()
        return 0

    lax.fori_loop(0, n_g - N_CP * CP_G, copy_tail, 0)

    c_ts.wait()
    c_lu.wait()
    c_lut.wait()

    def win_init(t, _):
        win_v[pl.ds(L * t, L)] = jnp.full((L,), -1, i32)
        return 0

    lax.fori_loop(0, n_g, win_init, 0)

    def win_scan(q, _):
        for u in range(UNROLL):
            k = UNROLL * q + u
            n16 = nodes_v[pl.ds(L * k, L)]
            loc = n16 - base_row
            m = plsc.bitcast(loc, jnp.uint32) < n_rows.astype(jnp.uint32)
            n_sel = jnp.where(m, n16, -1)
            _, last = plsc.scan_count(n_sel)
            plsc.store_scatter(win_v, [jnp.clip(loc, 0, n_rows - 1)],
                               iota16 + L * k, mask=m & last)
        return 0

    lax.fori_loop(0, BATCH // L // UNROLL, win_scan, 0)

    def comp(t, off):
        wv = win_v[pl.ds(L * t, L)]
        m = wv >= 0
        mi = m.astype(i32)
        c = plsc.cumsum(mi)
        tgt = jnp.maximum(off + c - 1, 0)
        plsc.store_scatter(mi_v, [tgt], wv, mask=m)
        plsc.store_scatter(mn_v, [tgt], base_row + L * t + iota16, mask=m)
        return off + c[L - 1]

    nuniq = lax.fori_loop(0, n_g, comp, jnp.int32(0))
    n_chunks = (nuniq + CHUNK_ROWS - 1) // CHUNK_ROWS

    @pl.when(nuniq > 0)
    def _():
        lastn = plsc.load_gather(mn_v, [jnp.full((L,), nuniq - 1, i32)])
        lasti = plsc.load_gather(mi_v, [jnp.full((L,), nuniq - 1, i32)])
        for t in range(VPC):
            plsc.store_scatter(mn_v, [nuniq + L * t + iota16], lastn)
            plsc.store_scatter(mi_v, [nuniq + L * t + iota16], lasti)

    def prep(k, _):
        n16 = mn_v[pl.ds(L * k, L)]
        wv = mi_v[pl.ds(L * k, L)]
        plsc.store_scatter(
            mnc_v, [jnp.full((L,), k // VPC, i32), (k % VPC) * L + iota16],
            n16)
        tsv = plsc.load_gather(ts_v, [wv])
        plsc.store_scatter(lu_v, [n16 - base_row], tsv)
        return 0

    lax.fori_loop(0, n_chunks * VPC, prep, 0)

    def pass_b(j, _):
        pltpu.async_copy(
            val_hbm.at[mi_v.at[pl.ds(CHUNK_ROWS * j, CHUNK_ROWS)]],
            rows_a.at[pl.ds(0, CHUNK_ROWS)], sem_a).wait()
        pltpu.async_copy(rows_a.at[pl.ds(0, CHUNK_ROWS)],
                         newmem_hbm.at[mnc_v.at[j]], sem_oa).wait()
        return 0

    lax.fori_loop(0, n_chunks, pass_b, 0)

    pltpu.async_copy(lu_v.at[pl.ds(0, MIN_ROWS)],
                     newlu_hbm.at[pl.ds(base_row, MIN_ROWS)], sem_oa).wait()

    @pl.when(n_g > GPW)
    def _():
        pltpu.async_copy(lu_v.at[pl.ds(MIN_ROWS, GRANULE)],
                         newlu_hbm.at[pl.ds(base_row + MIN_ROWS, GRANULE)],
                         sem_ob).wait()


@jax.jit
def kernel(memory, last_update, values, ts, nodes):
    mesh = plsc.VectorSubcoreMesh(core_axis_name="c", subcore_axis_name="s",
                                  num_cores=NC, num_subcores=NS)
    out_type = (
        jax.ShapeDtypeStruct((ROWS, DIM), jnp.float32),
        jax.ShapeDtypeStruct((ROWS,), jnp.float32),
        jax.ShapeDtypeStruct((BATCH, DIM), jnp.float32),
    )
    scratch = [
        pltpu.VMEM((BATCH,), jnp.int32),
        pltpu.VMEM((BATCH,), jnp.float32),
        pltpu.VMEM((MAX_OWN + CHUNK_ROWS,), jnp.int32),
        pltpu.VMEM((MAX_OWN + CHUNK_ROWS,), jnp.int32),
        pltpu.VMEM((MAX_CHUNKS, CHUNK_ROWS), jnp.int32),
        pltpu.VMEM((MAX_OWN,), jnp.int32),
        pltpu.VMEM((MAX_OWN,), jnp.float32),
        pltpu.VMEM((CP_ROWS, DIM), jnp.float32),
        pltpu.VMEM((CP_ROWS, DIM), jnp.float32),
        pltpu.SemaphoreType.DMA,
        pltpu.SemaphoreType.DMA,
        pltpu.SemaphoreType.DMA,
        pltpu.SemaphoreType.DMA,
        pltpu.SemaphoreType.DMA,
    ]
    cp = pltpu.CompilerParams(needs_layout_passes=False)
    f = pl.kernel(_body, out_type=out_type, mesh=mesh, scratch_types=scratch,
                  compiler_params=cp)
    return f(memory, last_update, values.astype(jnp.float32),
             ts.astype(jnp.float32), nodes.astype(jnp.int32))
